# baseline (speedup 1.0000x reference)
import functools

import jax
import jax.numpy as jnp
from jax import lax
from jax.experimental import pallas as pl
from jax.experimental.pallas import tpu as pltpu
from jax.experimental.pallas import tpu_sc as plsc

NETP = 16
ECH = 512
K = 128
NSC = 2
NSUB = 16


def _dot(a, b, ca, cb):
    return lax.dot_general(a, b, (((ca,), (cb,)), ((), ())),
                           preferred_element_type=jnp.float32)


def _a_body(nn_ref, src_ref, dst_ref, et_ref, nf_ref, Wp_ref, bp_ref,
            mW0_ref, h_ref, hW_ref, Et_ref, gs_ref, gd_ref, *, MAXN):
    b = pl.program_id(0)
    n = nn_ref[b]
    MAXE = src_ref.shape[2]
    NC = MAXE // ECH

    h = _dot(nf_ref[0], Wp_ref[...], 1, 1) + bp_ref[0]
    h_ref[0] = h
    hW_ref[0] = _dot(h, mW0_ref[...], 1, 1)

    src = src_ref[0, 0]
    dst = dst_ref[0, 0]
    emask = (src < n) & (dst < n)
    gs_ref[0, 0] = src + b * MAXN
    local = b - (b // 4) * 4
    gd_ref[0, 0] = jnp.where(emask, dst + local * MAXN, 4 * MAXN)

    iota_t = lax.broadcasted_iota(jnp.int32, (ECH, NETP), 1)
    iota_n = lax.broadcasted_iota(jnp.int32, (ECH, MAXN), 1)

    def chunk(c, carry):
        off, acc = carry
        off = pl.multiple_of(off, ECH)
        d = dst_ref[0, 0, pl.ds(off, ECH)].reshape(ECH, 1)
        s = src_ref[0, 0, pl.ds(off, ECH)].reshape(ECH, 1)
        t = et_ref[0, 0, pl.ds(off, ECH)].reshape(ECH, 1)
        em = (s < n) & (d < n)
        ohd = ((d == iota_n) & em).astype(jnp.float32)
        oht = (t == iota_t).astype(jnp.float32)
        return off + ECH, acc + _dot(ohd, oht, 0, 0)

    _, Et = lax.fori_loop(0, NC, chunk,
                          (jnp.int32(0), jnp.zeros((MAXN, NETP), jnp.float32)))
    Et_ref[0] = Et


def _sc_messages(hW_flat, gsrc2d, gdst2d, zrows, *, MAXN, DH):
    NROW = gsrc2d.shape[0]
    NCH = NROW // (NSC * NSUB)
    LROW = 4 * MAXN
    ZPT = LROW // NSUB + K
    mesh = plsc.VectorSubcoreMesh(core_axis_name="c", subcore_axis_name="s")

    NB = 2

    @functools.partial(
        pl.kernel, mesh=mesh,
        out_type=jax.ShapeDtypeStruct((NSC * LROW, DH), jnp.float32),
        scratch_types=(
            [pltpu.VMEM((K, DH), jnp.float32)] * NB
            + [pltpu.VMEM((NCH, K), jnp.int32)] * 2
            + [pltpu.VMEM_SHARED((LROW + NSUB * K, DH), jnp.float32)]
            + [pltpu.SemaphoreType.DMA] * (2 * NB)
        ),
    )
    def k(hW, gsrc, gdst, zr, out, *rest):
        bufs = rest[:NB]
        isrc, idst, msgs = rest[NB], rest[NB + 1], rest[NB + 2]
        gsems = rest[NB + 3:NB + 3 + NB]
        ssems = rest[NB + 3 + NB:]
        c = lax.axis_index("c")
        s = lax.axis_index("s")
        row0 = (c * NSUB + s) * NCH
        pltpu.sync_copy(gsrc.at[pl.ds(row0, NCH)], isrc)
        pltpu.sync_copy(gdst.at[pl.ds(row0, NCH)], idst)
        nz = ZPT // K
        for z in range(nz):
            pltpu.sync_copy(zr, msgs.at[pl.ds(s * ZPT + z * K, K)])
        plsc.subcore_barrier()

        def gath_start(j, b):
            pltpu.async_copy(hW.at[isrc.at[j]], bufs[b], gsems[b])

        def gath_wait(j, b):
            pltpu.make_async_copy(hW.at[isrc.at[j]], bufs[b], gsems[b]).wait()

        def scat_start(j, b):
            pltpu.async_copy(bufs[b], msgs.at[idst.at[j]], ssems[b], add=True)

        def scat_wait(j, b):
            pltpu.make_async_copy(bufs[b], msgs.at[idst.at[j]],
                                  ssems[b]).wait()

        for b in range(NB):
            gath_start(b, b)

        def body(i, carry):
            j0 = NB * i
            for b in range(NB):
                gath_wait(j0 + b, b)
                scat_start(j0 + b, b)
            for b in range(NB):
                @pl.when(j0 + b + NB < NCH)
                def _(b=b, j0=j0):
                    scat_wait(j0 + b, b)
                    gath_start(j0 + b + NB, b)
            return carry

        lax.fori_loop(0, NCH // NB, body, jnp.int32(0))
        for b in range(NB):
            scat_wait(NCH - NB + b, b)
        plsc.subcore_barrier()
        wpt = LROW // NSUB
        pltpu.sync_copy(msgs.at[pl.ds(s * wpt, wpt)],
                        out.at[pl.ds(c * LROW + s * wpt, wpt)])

    return k(hW_flat, gsrc2d, gdst2d, zrows)


def _d_body(nn_ref, h_ref, ms_ref, Et_ref, tab_ref, mb_ref, Wih_ref, bih_ref,
            Whh_ref, bhh_ref, mWn_ref, hn_ref, hWn_ref, *, MAXN, last):
    b = pl.program_id(0)
    n = nn_ref[b]
    DH = h_ref.shape[2]

    h = h_ref[0]
    Et = Et_ref[0]
    deg = jnp.sum(Et, axis=1, keepdims=True)
    he = jnp.sum(deg) > 0.5
    x = ms_ref[0] + _dot(Et, tab_ref[...], 1, 0) + deg * mb_ref[0]

    gi = _dot(x, Wih_ref[...], 1, 1) + bih_ref[0]
    gh = _dot(h, Whh_ref[...], 1, 1) + bhh_ref[0]
    r = jax.nn.sigmoid(gi[:, :DH] + gh[:, :DH])
    z = jax.nn.sigmoid(gi[:, DH:2 * DH] + gh[:, DH:2 * DH])
    ng = jnp.tanh(gi[:, 2 * DH:] + r * gh[:, 2 * DH:])
    gout = (1.0 - z) * ng + z * h

    valid = lax.broadcasted_iota(jnp.int32, (MAXN, 1), 0) < n
    hn = jnp.where(valid & he, gout, h)
    if last:
        hn_ref[0, 0] = jnp.sum(jnp.where(valid, hn, 0.0), axis=0)
    else:
        hn_ref[0] = hn
        hWn_ref[0] = _dot(hn, mWn_ref[...], 1, 1)


def _run(node_features, edge_index, edge_type, num_nodes, W_proj, b_proj,
         msg_W, msg_b, edge_tab, gru_Wih, gru_bih, gru_Whh, gru_bhh):
    B, MAXN, DF = node_features.shape
    L, DH, _ = msg_W.shape
    MAXE = edge_index.shape[2]

    src = edge_index[:, 0, :].reshape(B, 1, MAXE)
    dst = edge_index[:, 1, :].reshape(B, 1, MAXE)
    net = edge_tab.shape[1] - 1
    et = jnp.clip(edge_type, 0, net).reshape(B, 1, MAXE)
    tab = jnp.pad(edge_tab.astype(jnp.float32),
                  ((0, 0), (0, NETP - edge_tab.shape[1]), (0, 0)))
    nn = num_nodes.astype(jnp.int32)
    f32 = lambda a: a.astype(jnp.float32)

    full = lambda a: pl.BlockSpec(a.shape, lambda b: (0,) * a.ndim)
    smem = pl.BlockSpec(memory_space=pltpu.SMEM)
    eblk = pl.BlockSpec((1, 1, MAXE), lambda b: (b, 0, 0))
    nblk = pl.BlockSpec((1, MAXN, DH), lambda b: (b, 0, 0))

    h, hW, Et, gs, gd = pl.pallas_call(
        functools.partial(_a_body, MAXN=MAXN),
        grid=(B,),
        in_specs=[smem, eblk, eblk, eblk,
                  pl.BlockSpec((1, MAXN, DF), lambda b: (b, 0, 0)),
                  full(W_proj), pl.BlockSpec((1, DH), lambda b: (0, 0)),
                  full(msg_W[0])],
        out_specs=[nblk, nblk,
                   pl.BlockSpec((1, MAXN, NETP), lambda b: (b, 0, 0)),
                   eblk, eblk],
        out_shape=[jax.ShapeDtypeStruct((B, MAXN, DH), jnp.float32),
                   jax.ShapeDtypeStruct((B, MAXN, DH), jnp.float32),
                   jax.ShapeDtypeStruct((B, MAXN, NETP), jnp.float32),
                   jax.ShapeDtypeStruct((B, 1, MAXE), jnp.int32),
                   jax.ShapeDtypeStruct((B, 1, MAXE), jnp.int32)],
    )(nn, src, dst, et, f32(node_features), f32(W_proj),
      f32(b_proj).reshape(1, DH), f32(msg_W[0]))

    gs2 = gs.reshape(B * MAXE // K, K)
    gd2 = gd.reshape(B * MAXE // K, K)
    zrows = jnp.zeros((K, DH), jnp.float32)

    for l in range(L):
        last = l == L - 1
        msgs = _sc_messages(hW.reshape(B * MAXN, DH), gs2, gd2, zrows,
                            MAXN=MAXN, DH=DH).reshape(B, MAXN, DH)
        mWn = f32(msg_W[(l + 1) % L])
        if last:
            outs = [pl.BlockSpec((1, 1, DH), lambda b: (b, 0, 0)),
                    pl.BlockSpec((1, 1, DH), lambda b: (b, 0, 0))]
            oshape = [jax.ShapeDtypeStruct((B, 1, DH), jnp.float32),
                      jax.ShapeDtypeStruct((B, 1, DH), jnp.float32)]
        else:
            outs = [nblk, nblk]
            oshape = [jax.ShapeDtypeStruct((B, MAXN, DH), jnp.float32),
                      jax.ShapeDtypeStruct((B, MAXN, DH), jnp.float32)]
        h, hW = pl.pallas_call(
            functools.partial(_d_body, MAXN=MAXN, last=last),
            grid=(B,),
            in_specs=[smem, nblk, nblk,
                      pl.BlockSpec((1, MAXN, NETP), lambda b: (b, 0, 0)),
                      full(tab[l]), pl.BlockSpec((1, DH), lambda b: (0, 0)),
                      full(gru_Wih[l]), pl.BlockSpec((1, 3 * DH), lambda b: (0, 0)),
                      full(gru_Whh[l]), pl.BlockSpec((1, 3 * DH), lambda b: (0, 0)),
                      full(mWn)],
            out_specs=outs,
            out_shape=oshape,
        )(nn, h, msgs, Et, tab[l], f32(msg_b[l]).reshape(1, DH),
          f32(gru_Wih[l]), f32(gru_bih[l]).reshape(1, 3 * DH),
          f32(gru_Whh[l]), f32(gru_bhh[l]).reshape(1, 3 * DH), mWn)

    return h.reshape(B, DH)


def kernel(node_features, edge_index, edge_type, num_nodes, W_proj, b_proj,
           msg_W, msg_b, edge_tab, gru_Wih, gru_bih, gru_Whh, gru_bhh):
    edge_index = edge_index.astype(jnp.int32)
    edge_type = edge_type.astype(jnp.int32)
    num_nodes = num_nodes.astype(jnp.int32)
    with jax.enable_x64(False):
        out = _run(node_features, edge_index, edge_type, num_nodes, W_proj,
                   b_proj, msg_W, msg_b, edge_tab, gru_Wih, gru_bih,
                   gru_Whh, gru_bhh)
    return out.astype(jnp.float64)


# speedup vs baseline: 1.0338x; 1.0338x over previous
import functools

import jax
import jax.numpy as jnp
from jax import lax
from jax.experimental import pallas as pl
from jax.experimental.pallas import tpu as pltpu
from jax.experimental.pallas import tpu_sc as plsc

NETP = 16
ECH = 512
K = 128
NSC = 2
NSUB = 16


def _dot(a, b, ca, cb):
    return lax.dot_general(a, b, (((ca,), (cb,)), ((), ())),
                           preferred_element_type=jnp.float32)


def _a_body(nn_ref, src_ref, dst_ref, et_ref, nf_ref, Wp_ref, bp_ref,
            mW0_ref, h_ref, hW_ref, Et_ref, gs_ref, gd_ref, *, MAXN):
    b = pl.program_id(0)
    n = nn_ref[b]
    MAXE = src_ref.shape[2]
    NC = MAXE // ECH

    h = _dot(nf_ref[0], Wp_ref[...], 1, 1) + bp_ref[0]
    h_ref[0] = h
    hW_ref[0] = _dot(h, mW0_ref[...], 1, 1)

    src = src_ref[0, 0]
    dst = dst_ref[0, 0]
    emask = (src < n) & (dst < n)
    gs_ref[0, 0] = src + b * MAXN
    local = b - (b // 4) * 4
    gd_ref[0, 0] = jnp.where(emask, dst + local * MAXN, 4 * MAXN)

    iota_t = lax.broadcasted_iota(jnp.int32, (ECH, NETP), 1)
    iota_n = lax.broadcasted_iota(jnp.int32, (ECH, MAXN), 1)

    def chunk(c, carry):
        off, acc = carry
        off = pl.multiple_of(off, ECH)
        d = dst_ref[0, 0, pl.ds(off, ECH)].reshape(ECH, 1)
        s = src_ref[0, 0, pl.ds(off, ECH)].reshape(ECH, 1)
        t = et_ref[0, 0, pl.ds(off, ECH)].reshape(ECH, 1)
        em = (s < n) & (d < n)
        ohd = ((d == iota_n) & em).astype(jnp.float32)
        oht = (t == iota_t).astype(jnp.float32)
        return off + ECH, acc + _dot(ohd, oht, 0, 0)

    _, Et = lax.fori_loop(0, NC, chunk,
                          (jnp.int32(0), jnp.zeros((MAXN, NETP), jnp.float32)))
    Et_ref[0] = Et


def _sc_messages(hW_flat, gsrc2d, gdst2d, zrows, *, MAXN, DH):
    NROW = gsrc2d.shape[0]
    NCH = NROW // (NSC * NSUB)
    LROW = 4 * MAXN
    ZPT = LROW // NSUB + K
    mesh = plsc.VectorSubcoreMesh(core_axis_name="c", subcore_axis_name="s")

    NG = NCH

    @functools.partial(
        pl.kernel, mesh=mesh,
        out_type=jax.ShapeDtypeStruct((NSC * LROW, DH), jnp.float32),
        scratch_types=[
            pltpu.VMEM((K, DH), jnp.float32),
            pltpu.VMEM((K, DH), jnp.float32),
            pltpu.VMEM((NCH, K), jnp.int32),
            pltpu.VMEM((NCH, K), jnp.int32),
            pltpu.VMEM_SHARED((LROW + NSUB * K, DH), jnp.float32),
            pltpu.SemaphoreType.DMA,
            pltpu.SemaphoreType.DMA,
        ],
    )
    def k(hW, gsrc, gdst, zr, out, bufa, bufb, isrc, idst, msgs, sema, semb):
        c = lax.axis_index("c")
        s = lax.axis_index("s")
        row0 = (c * NSUB + s) * NCH
        pltpu.sync_copy(gsrc.at[pl.ds(row0, NCH)], isrc)
        pltpu.sync_copy(gdst.at[pl.ds(row0, NCH)], idst)
        nz = ZPT // K
        for z in range(nz):
            pltpu.sync_copy(zr, msgs.at[pl.ds(s * ZPT + z * K, K)])
        plsc.subcore_barrier()

        def gath(g, buf, sem):
            return pltpu.make_async_copy(hW.at[isrc.at[g]], buf, sem)

        def scat(g, buf):
            pltpu.sync_copy(buf, msgs.at[idst.at[g]], add=True)

        gath(0, bufa, sema).start()

        def body(i, carry):
            g0 = 2 * i
            gath(g0 + 1, bufb, semb).start()
            gath(g0, bufa, sema).wait()
            scat(g0, bufa)

            @pl.when(g0 + 2 < NG)
            def _():
                gath(g0 + 2, bufa, sema).start()

            gath(g0 + 1, bufb, semb).wait()
            scat(g0 + 1, bufb)
            return carry

        lax.fori_loop(0, NG // 2, body, jnp.int32(0))
        plsc.subcore_barrier()
        wpt = LROW // NSUB
        pltpu.sync_copy(msgs.at[pl.ds(s * wpt, wpt)],
                        out.at[pl.ds(c * LROW + s * wpt, wpt)])

    return k(hW_flat, gsrc2d, gdst2d, zrows)


def _d_body(nn_ref, h_ref, ms_ref, Et_ref, tab_ref, mb_ref, Wih_ref, bih_ref,
            Whh_ref, bhh_ref, mWn_ref, hn_ref, hWn_ref, *, MAXN, last):
    b = pl.program_id(0)
    n = nn_ref[b]
    DH = h_ref.shape[2]

    h = h_ref[0]
    Et = Et_ref[0]
    deg = jnp.sum(Et, axis=1, keepdims=True)
    he = jnp.sum(deg) > 0.5
    x = ms_ref[0] + _dot(Et, tab_ref[...], 1, 0) + deg * mb_ref[0]

    gi = _dot(x, Wih_ref[...], 1, 1) + bih_ref[0]
    gh = _dot(h, Whh_ref[...], 1, 1) + bhh_ref[0]
    r = jax.nn.sigmoid(gi[:, :DH] + gh[:, :DH])
    z = jax.nn.sigmoid(gi[:, DH:2 * DH] + gh[:, DH:2 * DH])
    ng = jnp.tanh(gi[:, 2 * DH:] + r * gh[:, 2 * DH:])
    gout = (1.0 - z) * ng + z * h

    valid = lax.broadcasted_iota(jnp.int32, (MAXN, 1), 0) < n
    hn = jnp.where(valid & he, gout, h)
    if last:
        hn_ref[0, 0] = jnp.sum(jnp.where(valid, hn, 0.0), axis=0)
    else:
        hn_ref[0] = hn
        hWn_ref[0] = _dot(hn, mWn_ref[...], 1, 1)


def _run(node_features, edge_index, edge_type, num_nodes, W_proj, b_proj,
         msg_W, msg_b, edge_tab, gru_Wih, gru_bih, gru_Whh, gru_bhh):
    B, MAXN, DF = node_features.shape
    L, DH, _ = msg_W.shape
    MAXE = edge_index.shape[2]

    src = edge_index[:, 0, :].reshape(B, 1, MAXE)
    dst = edge_index[:, 1, :].reshape(B, 1, MAXE)
    net = edge_tab.shape[1] - 1
    et = jnp.clip(edge_type, 0, net).reshape(B, 1, MAXE)
    tab = jnp.pad(edge_tab.astype(jnp.float32),
                  ((0, 0), (0, NETP - edge_tab.shape[1]), (0, 0)))
    nn = num_nodes.astype(jnp.int32)
    f32 = lambda a: a.astype(jnp.float32)

    full = lambda a: pl.BlockSpec(a.shape, lambda b: (0,) * a.ndim)
    smem = pl.BlockSpec(memory_space=pltpu.SMEM)
    eblk = pl.BlockSpec((1, 1, MAXE), lambda b: (b, 0, 0))
    nblk = pl.BlockSpec((1, MAXN, DH), lambda b: (b, 0, 0))

    h, hW, Et, gs, gd = pl.pallas_call(
        functools.partial(_a_body, MAXN=MAXN),
        grid=(B,),
        in_specs=[smem, eblk, eblk, eblk,
                  pl.BlockSpec((1, MAXN, DF), lambda b: (b, 0, 0)),
                  full(W_proj), pl.BlockSpec((1, DH), lambda b: (0, 0)),
                  full(msg_W[0])],
        out_specs=[nblk, nblk,
                   pl.BlockSpec((1, MAXN, NETP), lambda b: (b, 0, 0)),
                   eblk, eblk],
        out_shape=[jax.ShapeDtypeStruct((B, MAXN, DH), jnp.float32),
                   jax.ShapeDtypeStruct((B, MAXN, DH), jnp.float32),
                   jax.ShapeDtypeStruct((B, MAXN, NETP), jnp.float32),
                   jax.ShapeDtypeStruct((B, 1, MAXE), jnp.int32),
                   jax.ShapeDtypeStruct((B, 1, MAXE), jnp.int32)],
    )(nn, src, dst, et, f32(node_features), f32(W_proj),
      f32(b_proj).reshape(1, DH), f32(msg_W[0]))

    gs2 = gs.reshape(B * MAXE // K, K)
    gd2 = gd.reshape(B * MAXE // K, K)
    zrows = jnp.zeros((K, DH), jnp.float32)

    for l in range(L):
        last = l == L - 1
        msgs = _sc_messages(hW.reshape(B * MAXN, DH), gs2, gd2, zrows,
                            MAXN=MAXN, DH=DH).reshape(B, MAXN, DH)
        mWn = f32(msg_W[(l + 1) % L])
        if last:
            outs = [pl.BlockSpec((1, 1, DH), lambda b: (b, 0, 0)),
                    pl.BlockSpec((1, 1, DH), lambda b: (b, 0, 0))]
            oshape = [jax.ShapeDtypeStruct((B, 1, DH), jnp.float32),
                      jax.ShapeDtypeStruct((B, 1, DH), jnp.float32)]
        else:
            outs = [nblk, nblk]
            oshape = [jax.ShapeDtypeStruct((B, MAXN, DH), jnp.float32),
                      jax.ShapeDtypeStruct((B, MAXN, DH), jnp.float32)]
        h, hW = pl.pallas_call(
            functools.partial(_d_body, MAXN=MAXN, last=last),
            grid=(B,),
            in_specs=[smem, nblk, nblk,
                      pl.BlockSpec((1, MAXN, NETP), lambda b: (b, 0, 0)),
                      full(tab[l]), pl.BlockSpec((1, DH), lambda b: (0, 0)),
                      full(gru_Wih[l]), pl.BlockSpec((1, 3 * DH), lambda b: (0, 0)),
                      full(gru_Whh[l]), pl.BlockSpec((1, 3 * DH), lambda b: (0, 0)),
                      full(mWn)],
            out_specs=outs,
            out_shape=oshape,
        )(nn, h, msgs, Et, tab[l], f32(msg_b[l]).reshape(1, DH),
          f32(gru_Wih[l]), f32(gru_bih[l]).reshape(1, 3 * DH),
          f32(gru_Whh[l]), f32(gru_bhh[l]).reshape(1, 3 * DH), mWn)

    return h.reshape(B, DH)


def kernel(node_features, edge_index, edge_type, num_nodes, W_proj, b_proj,
           msg_W, msg_b, edge_tab, gru_Wih, gru_bih, gru_Whh, gru_bhh):
    edge_index = edge_index.astype(jnp.int32)
    edge_type = edge_type.astype(jnp.int32)
    num_nodes = num_nodes.astype(jnp.int32)
    with jax.enable_x64(False):
        out = _run(node_features, edge_index, edge_type, num_nodes, W_proj,
                   b_proj, msg_W, msg_b, edge_tab, gru_Wih, gru_bih,
                   gru_Whh, gru_bhh)
    return out.astype(jnp.float64)
